# baseline (speedup 1.0000x reference)
import functools

import jax
import jax.numpy as jnp
from jax import lax
from jax.experimental import pallas as pl
from jax.experimental.pallas import tpu as pltpu
from jax.experimental.pallas import tpu_sc as plsc

NC = 2
NS = 16
CHUNK = 128
SW = 16
NR = 2
F32 = jnp.float32

_SELU_ALPHA = 1.6732632423543772848170429916717
_SELU_SCALE = 1.0507009873554804934193349852946


def _mesh():
    return plsc.VectorSubcoreMesh(
        core_axis_name="c", subcore_axis_name="s", num_cores=NC, num_subcores=NS)


def _make_degree_kernel(npad, n_chunks):
    rows_per_tile = npad // NS

    @functools.partial(
        pl.kernel,
        out_type=jax.ShapeDtypeStruct((NC, npad, SW), F32),
        mesh=_mesh(),
        compiler_params=pltpu.CompilerParams(use_tc_tiling_on_sc=False),
        scratch_types=[
            pltpu.VMEM((n_chunks, CHUNK), jnp.int32),
            pltpu.VMEM((CHUNK, SW), F32),
            pltpu.VMEM((rows_per_tile, SW), F32),
            pltpu.VMEM_SHARED((npad, SW), F32),
        ],
    )
    def deg_kernel(edges_hbm, deg_out, idx_v, ones_v, stage_v, acc_sh):
        c = lax.axis_index("c")
        s = lax.axis_index("s")

        def zrow(i, _):
            stage_v[i, :] = jnp.zeros((SW,), F32)
            return 0
        lax.fori_loop(0, rows_per_tile, zrow, 0)
        pltpu.sync_copy(stage_v, acc_sh.at[pl.ds(s * rows_per_tile, rows_per_tile)])

        def orow(i, _):
            ones_v[i, :] = jnp.full((SW,), 1.0, F32)
            return 0
        lax.fori_loop(0, CHUNK, orow, 0)
        pltpu.sync_copy(edges_hbm.at[c, s], idx_v)
        plsc.subcore_barrier()

        def step(j, _):
            pltpu.sync_copy(ones_v, acc_sh.at[idx_v.at[j]], add=True)
            return 0
        lax.fori_loop(0, n_chunks, step, 0)
        plsc.subcore_barrier()
        pltpu.sync_copy(acc_sh.at[pl.ds(s * rows_per_tile, rows_per_tile)], stage_v)
        pltpu.sync_copy(stage_v, deg_out.at[c, pl.ds(s * rows_per_tile, rows_per_tile)])

    return deg_kernel


def _make_agg_kernel(npad, n_chunks, dw):
    nspan = npad // NR
    rows_per_tile = nspan // NS
    wb = 64
    acc_rows = nspan + 8
    trash = nspan
    NBUF = 2
    assert n_chunks % NBUF == 0

    @functools.partial(
        pl.kernel,
        out_type=jax.ShapeDtypeStruct((NC * npad, dw), F32),
        mesh=_mesh(),
        compiler_params=pltpu.CompilerParams(use_tc_tiling_on_sc=False),
        scratch_types=[
            pltpu.VMEM((n_chunks, CHUNK), jnp.int32),
            pltpu.VMEM((n_chunks, CHUNK), jnp.int32),
            [pltpu.VMEM((CHUNK, dw), F32) for _ in range(NBUF)],
            pltpu.VMEM_SHARED((acc_rows, dw), F32),
            [pltpu.SemaphoreType.DMA for _ in range(NBUF)],
            [pltpu.SemaphoreType.DMA for _ in range(NBUF)],
        ],
    )
    def agg_kernel(xs_hbm, send_hbm, recv_hbm, agg_out,
                   sidx_v, rloc_v, rows, acc_sh, gsem, ssem):
        c = lax.axis_index("c")
        s = lax.axis_index("s")

        pltpu.sync_copy(send_hbm.at[s], sidx_v)
        off = jnp.broadcast_to(c * npad, (16,)).astype(jnp.int32)

        def adj(j, _):
            for k in range(CHUNK // 16):
                sl = pl.ds(k * 16, 16)
                sidx_v[j, sl] = sidx_v[j, sl] + off
            return 0
        lax.fori_loop(0, n_chunks, adj, 0)

        for r in range(NR):
            base = r * nspan
            basev = jnp.broadcast_to(base, (16,)).astype(jnp.int32)
            spanv = jnp.broadcast_to(nspan, (16,)).astype(jnp.int32)
            trashv = jnp.broadcast_to(trash, (16,)).astype(jnp.int32)

            def zrow(i, _):
                for k in range(dw // 16):
                    rows[0][i, pl.ds(k * 16, 16)] = jnp.zeros((16,), F32)
                return 0
            lax.fori_loop(0, wb, zrow, 0)
            for k in range(rows_per_tile // wb):
                pltpu.sync_copy(
                    rows[0].at[pl.ds(0, wb)],
                    acc_sh.at[pl.ds(s * rows_per_tile + k * wb, wb)])

            @pl.when(s == 0)
            def _():
                pltpu.sync_copy(rows[0].at[pl.ds(0, 8)],
                                acc_sh.at[pl.ds(trash, 8)])

            pltpu.sync_copy(recv_hbm.at[s], rloc_v)

            def loc(j, _):
                for k in range(CHUNK // 16):
                    sl = pl.ds(k * 16, 16)
                    v = rloc_v[j, sl] - basev
                    ok = (v >= 0) & (v < spanv)
                    rloc_v[j, sl] = jnp.where(ok, v, trashv)
                return 0
            lax.fori_loop(0, n_chunks, loc, 0)
            plsc.subcore_barrier()

            for b in range(NBUF):
                pltpu.async_copy(xs_hbm.at[sidx_v.at[b]], rows[b], gsem[b])

            def step(p, _):
                j0 = p * NBUF
                for b in range(NBUF):
                    pltpu.make_async_copy(
                        xs_hbm.at[sidx_v.at[j0 + b]], rows[b], gsem[b]).wait()
                    pltpu.make_async_copy(
                        rows[b], acc_sh.at[rloc_v.at[j0 + b]], ssem[b]
                    ).start(add=True)
                for b in range(NBUF):
                    pltpu.make_async_copy(
                        rows[b], acc_sh.at[rloc_v.at[j0 + b]], ssem[b]).wait()

                    @pl.when(j0 + NBUF + b < n_chunks)
                    def _():
                        pltpu.async_copy(
                            xs_hbm.at[sidx_v.at[j0 + NBUF + b]], rows[b], gsem[b])
                return 0
            lax.fori_loop(0, n_chunks // NBUF, step, 0)
            plsc.subcore_barrier()

            for k in range(rows_per_tile // wb):
                pltpu.sync_copy(
                    acc_sh.at[pl.ds(s * rows_per_tile + k * wb, wb)],
                    rows[0].at[pl.ds(0, wb)])
                pltpu.sync_copy(
                    rows[0].at[pl.ds(0, wb)],
                    agg_out.at[pl.ds(
                        c * npad + base + s * rows_per_tile + k * wb, wb)])

    return agg_kernel


def _scale_rows(x2, sdeg, npad, d, blk):
    nblk = npad // blk
    grid = (2 * nblk,)

    def body(x_ref, deg_ref, xs_ref):
        dsv = lax.rsqrt(jnp.maximum(deg_ref[...], 1.0))
        xs_ref[:, :d] = x_ref[...] * dsv
        xs_ref[:, d:] = jnp.broadcast_to(dsv, (blk, SW))

    return pl.pallas_call(
        body,
        grid=grid,
        in_specs=[
            pl.BlockSpec((blk, d), lambda g: (g, 0)),
            pl.BlockSpec((blk, 1), lambda g: (lax.rem(g, nblk), 0)),
        ],
        out_specs=pl.BlockSpec((blk, d + SW), lambda g: (g, 0)),
        out_shape=jax.ShapeDtypeStruct((2 * npad, d + SW), F32),
    )(x2, sdeg)


def _dense_selu(agg2, rdeg, W, b2, npad, d, h, n_true, blk):
    nblk = npad // blk
    grid = (2 * nblk,)

    def body(agg_ref, deg_ref, w_ref, b_ref, nodes_ref, colsum_ref):
        g = pl.program_id(0)
        aggm = agg_ref[...]
        u = jnp.dot(aggm[:, :d], w_ref[...], preferred_element_type=F32)
        dr = lax.rsqrt(jnp.maximum(deg_ref[...], 1.0))
        t = aggm[:, d:d + 1]
        hh = u * dr + (dr * t) * b_ref[...]
        nodes = _SELU_SCALE * jnp.where(hh > 0, hh, _SELU_ALPHA * (lax.exp(hh) - 1.0))
        nodes_ref[...] = nodes

        @pl.when(g == 0)
        def _():
            colsum_ref[...] = jnp.zeros((1, h), F32)

        @pl.when(g < nblk)
        def _():
            row = lax.broadcasted_iota(jnp.int32, (blk, 1), 0) + g * blk
            masked = jnp.where(row < n_true, nodes, jnp.zeros((blk, h), F32))
            colsum_ref[...] += jnp.sum(masked, axis=0, keepdims=True)

    return pl.pallas_call(
        body,
        grid=grid,
        in_specs=[
            pl.BlockSpec((blk, d + SW), lambda g: (g, 0)),
            pl.BlockSpec((blk, 1), lambda g: (lax.rem(g, nblk), 0)),
            pl.BlockSpec((d, h), lambda g: (0, 0)),
            pl.BlockSpec((1, h), lambda g: (0, 0)),
        ],
        out_specs=[
            pl.BlockSpec((blk, h), lambda g: (g, 0)),
            pl.BlockSpec((1, h), lambda g: (0, 0)),
        ],
        out_shape=[
            jax.ShapeDtypeStruct((2 * npad, h), F32),
            jax.ShapeDtypeStruct((1, h), F32),
        ],
    )(agg2, rdeg, W, b2)


def _readout(nodes2, colsum, Wb, npad, h, n_true, blk):
    nblk = npad // blk
    grid = (2 * nblk,)

    def body(nodes_ref, colsum_ref, wb_ref, logits_ref, summary_ref, v_s):
        g = pl.program_id(0)

        @pl.when(g == 0)
        def _():
            sig = jax.nn.sigmoid(colsum_ref[...] * (1.0 / n_true))
            summary_ref[...] = sig
            v_s[...] = jnp.dot(wb_ref[...], jnp.reshape(sig, (h, 1)),
                               preferred_element_type=F32)

        logits_ref[...] = jnp.dot(nodes_ref[...], v_s[...],
                                  preferred_element_type=F32)

    return pl.pallas_call(
        body,
        grid=grid,
        in_specs=[
            pl.BlockSpec((blk, h), lambda g: (g, 0)),
            pl.BlockSpec((1, h), lambda g: (0, 0)),
            pl.BlockSpec((h, h), lambda g: (0, 0)),
        ],
        out_specs=[
            pl.BlockSpec((blk, 1), lambda g: (g, 0)),
            pl.BlockSpec((1, h), lambda g: (0, 0)),
        ],
        out_shape=[
            jax.ShapeDtypeStruct((2 * npad, 1), F32),
            jax.ShapeDtypeStruct((1, h), F32),
        ],
        scratch_shapes=[pltpu.VMEM((h, 1), F32)],
    )(nodes2, colsum, Wb)


def kernel(x, c_x, edge_index, W, b, Wb):
    n, d = x.shape
    h = W.shape[1]
    e = edge_index.shape[1]
    blk = 256
    npad = ((n + 1 + 2 * blk - 1) // (2 * blk)) * (2 * blk)
    echunk = NS * CHUNK * 2
    epad = ((e + echunk - 1) // echunk) * echunk
    n_chunks = epad // (NS * CHUNK)

    senders = edge_index[0].astype(jnp.int32)
    receivers = edge_index[1].astype(jnp.int32)
    pad_e = jnp.full((epad - e,), n, jnp.int32)
    sr = jnp.concatenate([senders, pad_e])
    rr = jnp.concatenate([receivers, pad_e])
    edges3 = jnp.stack([sr, rr]).reshape(2, NS, n_chunks, CHUNK)
    send_g = sr.reshape(NS, n_chunks, CHUNK)
    recv3 = rr.reshape(NS, n_chunks, CHUNK)

    deg2 = _make_degree_kernel(npad, n_chunks)(edges3)
    sdeg = deg2[0, :, 0:1]
    rdeg = deg2[1, :, 0:1]

    x2 = jnp.concatenate([
        jnp.pad(x.astype(F32), ((0, npad - n), (0, 0))),
        jnp.pad(c_x.astype(F32), ((0, npad - n), (0, 0))),
    ])
    xs2 = _scale_rows(x2, sdeg, npad, d, blk)

    agg2 = _make_agg_kernel(npad, n_chunks, d + SW)(xs2, send_g, recv3)

    b2 = jnp.reshape(b.astype(F32), (1, h))
    nodes2, colsum = _dense_selu(agg2, rdeg, W.astype(F32), b2, npad, d, h, n, blk)

    logits2, summary = _readout(nodes2, colsum, Wb.astype(F32), npad, h, n, blk)

    nodes1_o = nodes2[:n]
    nodes2_o = nodes2[npad:npad + n]
    logits = jnp.concatenate([logits2[:n, 0], logits2[npad:npad + n, 0]])
    return (nodes1_o, nodes2_o, summary[0]), logits


# speedup vs baseline: 1.1149x; 1.1149x over previous
import functools

import jax
import jax.numpy as jnp
from jax import lax
from jax.experimental import pallas as pl
from jax.experimental.pallas import tpu as pltpu
from jax.experimental.pallas import tpu_sc as plsc

NC = 2
NS = 16
CHUNK = 96
SW = 8
NR = 2
F32 = jnp.float32

_SELU_ALPHA = 1.6732632423543772848170429916717
_SELU_SCALE = 1.0507009873554804934193349852946


def _mesh():
    return plsc.VectorSubcoreMesh(
        core_axis_name="c", subcore_axis_name="s", num_cores=NC, num_subcores=NS)


def _make_degree_kernel(npad, n_chunks):
    rows_per_tile = npad // NS

    @functools.partial(
        pl.kernel,
        out_type=jax.ShapeDtypeStruct((NC, npad, SW), F32),
        mesh=_mesh(),
        compiler_params=pltpu.CompilerParams(use_tc_tiling_on_sc=False),
        scratch_types=[
            pltpu.VMEM((n_chunks, CHUNK), jnp.int32),
            pltpu.VMEM((CHUNK, SW), F32),
            pltpu.VMEM((rows_per_tile, SW), F32),
            pltpu.VMEM_SHARED((npad, SW), F32),
        ],
    )
    def deg_kernel(edges_hbm, deg_out, idx_v, ones_v, stage_v, acc_sh):
        c = lax.axis_index("c")
        s = lax.axis_index("s")

        def zrow(i, _):
            stage_v[i, :] = jnp.zeros((SW,), F32)
            return 0
        lax.fori_loop(0, rows_per_tile, zrow, 0)
        pltpu.sync_copy(stage_v, acc_sh.at[pl.ds(s * rows_per_tile, rows_per_tile)])

        def orow(i, _):
            ones_v[i, :] = jnp.full((SW,), 1.0, F32)
            return 0
        lax.fori_loop(0, CHUNK, orow, 0)
        pltpu.sync_copy(edges_hbm.at[c, s], idx_v)
        plsc.subcore_barrier()

        def step(j, _):
            pltpu.sync_copy(ones_v, acc_sh.at[idx_v.at[j]], add=True)
            return 0
        lax.fori_loop(0, n_chunks, step, 0)
        plsc.subcore_barrier()
        pltpu.sync_copy(acc_sh.at[pl.ds(s * rows_per_tile, rows_per_tile)], stage_v)
        pltpu.sync_copy(stage_v, deg_out.at[c, pl.ds(s * rows_per_tile, rows_per_tile)])

    return deg_kernel


def _make_agg_kernel(npad, n_chunks, dw):
    nspan = npad // NR
    rows_per_tile = nspan // NS
    wb = 64
    acc_rows = nspan + 8
    trash = nspan

    @functools.partial(
        pl.kernel,
        out_type=jax.ShapeDtypeStruct((NC * npad, dw), F32),
        mesh=_mesh(),
        compiler_params=pltpu.CompilerParams(use_tc_tiling_on_sc=False),
        scratch_types=[
            pltpu.VMEM((n_chunks, CHUNK), jnp.int32),
            pltpu.VMEM((n_chunks, CHUNK), jnp.int32),
            pltpu.VMEM((n_chunks, CHUNK), jnp.int32),
            pltpu.VMEM((CHUNK, dw), F32),
            pltpu.VMEM((CHUNK, dw), F32),
            pltpu.VMEM_SHARED((acc_rows, dw), F32),
            pltpu.SemaphoreType.DMA,
            pltpu.SemaphoreType.DMA,
        ],
    )
    def agg_kernel(xs_hbm, send_hbm, recv_hbm, agg_out,
                   sidx_v, ridx_v, rloc_v, rows_a, rows_b, acc_sh,
                   gsem_a, gsem_b):
        c = lax.axis_index("c")
        s = lax.axis_index("s")

        pltpu.sync_copy(send_hbm.at[s], sidx_v)
        pltpu.sync_copy(recv_hbm.at[s], ridx_v)
        off = jnp.broadcast_to(c * npad, (16,)).astype(jnp.int32)

        def adj(j, _):
            for k in range(CHUNK // 16):
                sl = pl.ds(k * 16, 16)
                sidx_v[j, sl] = sidx_v[j, sl] + off
            return 0
        lax.fori_loop(0, n_chunks, adj, 0)

        def zrow(i, _):
            for k in range(dw // 16):
                rows_a[i, pl.ds(k * 16, 16)] = jnp.zeros((16,), F32)
            if dw % 16:
                rows_a[i, pl.ds(16 * (dw // 16), dw % 16)] = jnp.zeros((dw % 16,), F32)
            return 0
        lax.fori_loop(0, CHUNK, zrow, 0)

        for r in range(NR):
            base = r * nspan
            basev = jnp.broadcast_to(base, (16,)).astype(jnp.int32)
            spanv = jnp.broadcast_to(nspan, (16,)).astype(jnp.int32)
            trashv = jnp.broadcast_to(trash, (16,)).astype(jnp.int32)

            for k in range(rows_per_tile // wb):
                pltpu.sync_copy(
                    rows_a.at[pl.ds(0, wb)],
                    acc_sh.at[pl.ds(s * rows_per_tile + k * wb, wb)])

            @pl.when(s == 0)
            def _():
                pltpu.sync_copy(rows_a.at[pl.ds(0, 8)],
                                acc_sh.at[pl.ds(trash, 8)])

            def loc(j, _):
                for k in range(CHUNK // 16):
                    sl = pl.ds(k * 16, 16)
                    v = ridx_v[j, sl] - basev
                    ok = (v >= 0) & (v < spanv)
                    rloc_v[j, sl] = jnp.where(ok, v, trashv)
                return 0
            lax.fori_loop(0, n_chunks, loc, 0)
            plsc.subcore_barrier()

            pltpu.async_copy(xs_hbm.at[sidx_v.at[0]], rows_a, gsem_a)

            def step(p, _):
                j = 2 * p
                pltpu.make_async_copy(
                    xs_hbm.at[sidx_v.at[j]], rows_a, gsem_a).wait()
                pltpu.async_copy(xs_hbm.at[sidx_v.at[j + 1]], rows_b, gsem_b)
                pltpu.sync_copy(rows_a, acc_sh.at[rloc_v.at[j]], add=True)
                pltpu.make_async_copy(
                    xs_hbm.at[sidx_v.at[j + 1]], rows_b, gsem_b).wait()

                @pl.when(j + 2 < n_chunks)
                def _():
                    pltpu.async_copy(xs_hbm.at[sidx_v.at[j + 2]], rows_a, gsem_a)
                pltpu.sync_copy(rows_b, acc_sh.at[rloc_v.at[j + 1]], add=True)
                return 0
            lax.fori_loop(0, n_chunks // 2, step, 0)
            plsc.subcore_barrier()

            for k in range(rows_per_tile // wb):
                pltpu.sync_copy(
                    acc_sh.at[pl.ds(s * rows_per_tile + k * wb, wb)],
                    rows_a.at[pl.ds(0, wb)])
                pltpu.sync_copy(
                    rows_a.at[pl.ds(0, wb)],
                    agg_out.at[pl.ds(
                        c * npad + base + s * rows_per_tile + k * wb, wb)])

            if r + 1 < NR:
                def zrow2(i, _):
                    for k in range(dw // 16):
                        rows_a[i, pl.ds(k * 16, 16)] = jnp.zeros((16,), F32)
                    if dw % 16:
                        rows_a[i, pl.ds(16 * (dw // 16), dw % 16)] = jnp.zeros((dw % 16,), F32)
                    return 0
                lax.fori_loop(0, wb, zrow2, 0)

    return agg_kernel


def _scale_rows(x2, sdeg, npad, d, blk):
    nblk = npad // blk
    grid = (2 * nblk,)

    def body(x_ref, deg_ref, xs_ref):
        dsv = lax.rsqrt(jnp.maximum(deg_ref[...], 1.0))
        xs_ref[:, :d] = x_ref[...] * dsv
        xs_ref[:, d:] = jnp.broadcast_to(dsv, (blk, SW))

    return pl.pallas_call(
        body,
        grid=grid,
        in_specs=[
            pl.BlockSpec((blk, d), lambda g: (g, 0)),
            pl.BlockSpec((blk, 1), lambda g: (lax.rem(g, nblk), 0)),
        ],
        out_specs=pl.BlockSpec((blk, d + SW), lambda g: (g, 0)),
        out_shape=jax.ShapeDtypeStruct((2 * npad, d + SW), F32),
    )(x2, sdeg)


def _dense_selu(agg2, rdeg, W, b2, npad, d, h, n_true, blk):
    nblk = npad // blk
    grid = (2 * nblk,)

    def body(agg_ref, deg_ref, w_ref, b_ref, nodes_ref, colsum_ref):
        g = pl.program_id(0)
        aggm = agg_ref[...]
        u = jnp.dot(aggm[:, :d], w_ref[...], preferred_element_type=F32)
        dr = lax.rsqrt(jnp.maximum(deg_ref[...], 1.0))
        t = aggm[:, d:d + 1]
        hh = u * dr + (dr * t) * b_ref[...]
        nodes = _SELU_SCALE * jnp.where(hh > 0, hh, _SELU_ALPHA * (lax.exp(hh) - 1.0))
        nodes_ref[...] = nodes

        @pl.when(g == 0)
        def _():
            colsum_ref[...] = jnp.zeros((1, h), F32)

        @pl.when(g < nblk)
        def _():
            row = lax.broadcasted_iota(jnp.int32, (blk, 1), 0) + g * blk
            masked = jnp.where(row < n_true, nodes, jnp.zeros((blk, h), F32))
            colsum_ref[...] += jnp.sum(masked, axis=0, keepdims=True)

    return pl.pallas_call(
        body,
        grid=grid,
        in_specs=[
            pl.BlockSpec((blk, d + SW), lambda g: (g, 0)),
            pl.BlockSpec((blk, 1), lambda g: (lax.rem(g, nblk), 0)),
            pl.BlockSpec((d, h), lambda g: (0, 0)),
            pl.BlockSpec((1, h), lambda g: (0, 0)),
        ],
        out_specs=[
            pl.BlockSpec((blk, h), lambda g: (g, 0)),
            pl.BlockSpec((1, h), lambda g: (0, 0)),
        ],
        out_shape=[
            jax.ShapeDtypeStruct((2 * npad, h), F32),
            jax.ShapeDtypeStruct((1, h), F32),
        ],
    )(agg2, rdeg, W, b2)


def _readout(nodes2, colsum, Wb, npad, h, n_true, blk):
    nblk = npad // blk
    grid = (2 * nblk,)

    def body(nodes_ref, colsum_ref, wb_ref, logits_ref, summary_ref, v_s):
        g = pl.program_id(0)

        @pl.when(g == 0)
        def _():
            sig = jax.nn.sigmoid(colsum_ref[...] * (1.0 / n_true))
            summary_ref[...] = sig
            v_s[...] = jnp.dot(wb_ref[...], jnp.reshape(sig, (h, 1)),
                               preferred_element_type=F32)

        logits_ref[...] = jnp.dot(nodes_ref[...], v_s[...],
                                  preferred_element_type=F32)

    return pl.pallas_call(
        body,
        grid=grid,
        in_specs=[
            pl.BlockSpec((blk, h), lambda g: (g, 0)),
            pl.BlockSpec((1, h), lambda g: (0, 0)),
            pl.BlockSpec((h, h), lambda g: (0, 0)),
        ],
        out_specs=[
            pl.BlockSpec((blk, 1), lambda g: (g, 0)),
            pl.BlockSpec((1, h), lambda g: (0, 0)),
        ],
        out_shape=[
            jax.ShapeDtypeStruct((2 * npad, 1), F32),
            jax.ShapeDtypeStruct((1, h), F32),
        ],
        scratch_shapes=[pltpu.VMEM((h, 1), F32)],
    )(nodes2, colsum, Wb)


def kernel(x, c_x, edge_index, W, b, Wb):
    n, d = x.shape
    h = W.shape[1]
    e = edge_index.shape[1]
    blk = 256
    npad = ((n + 1 + 2 * blk - 1) // (2 * blk)) * (2 * blk)
    echunk = NS * CHUNK * 2
    epad = ((e + echunk - 1) // echunk) * echunk
    n_chunks = epad // (NS * CHUNK)

    senders = edge_index[0].astype(jnp.int32)
    receivers = edge_index[1].astype(jnp.int32)
    pad_e = jnp.full((epad - e,), n, jnp.int32)
    sr = jnp.concatenate([senders, pad_e])
    rr = jnp.concatenate([receivers, pad_e])
    edges3 = jnp.stack([sr, rr]).reshape(2, NS, n_chunks, CHUNK)
    send_g = sr.reshape(NS, n_chunks, CHUNK)
    recv3 = rr.reshape(NS, n_chunks, CHUNK)

    deg2 = _make_degree_kernel(npad, n_chunks)(edges3)
    sdeg = deg2[0, :, 0:1]
    rdeg = deg2[1, :, 0:1]

    x2 = jnp.concatenate([
        jnp.pad(x.astype(F32), ((0, npad - n), (0, 0))),
        jnp.pad(c_x.astype(F32), ((0, npad - n), (0, 0))),
    ])
    xs2 = _scale_rows(x2, sdeg, npad, d, blk)

    agg2 = _make_agg_kernel(npad, n_chunks, d + SW)(xs2, send_g, recv3)

    b2 = jnp.reshape(b.astype(F32), (1, h))
    nodes2, colsum = _dense_selu(agg2, rdeg, W.astype(F32), b2, npad, d, h, n, blk)

    logits2, summary = _readout(nodes2, colsum, Wb.astype(F32), npad, h, n, blk)

    nodes1_o = nodes2[:n]
    nodes2_o = nodes2[npad:npad + n]
    logits = jnp.concatenate([logits2[:n, 0], logits2[npad:npad + n, 0]])
    return (nodes1_o, nodes2_o, summary[0]), logits


# speedup vs baseline: 1.3624x; 1.2220x over previous
import functools

import jax
import jax.numpy as jnp
from jax import lax
from jax.experimental import pallas as pl
from jax.experimental.pallas import tpu as pltpu
from jax.experimental.pallas import tpu_sc as plsc

NC = 2
NS = 16
CHUNK = 80
SW = 16
DEGW = 8
NR = 2
F32 = jnp.float32

_SELU_ALPHA = 1.6732632423543772848170429916717
_SELU_SCALE = 1.0507009873554804934193349852946


def _mesh():
    return plsc.VectorSubcoreMesh(
        core_axis_name="c", subcore_axis_name="s", num_cores=NC, num_subcores=NS)


def _make_degree_kernel(npad, n_chunks):
    rows_per_tile = npad // NS

    @functools.partial(
        pl.kernel,
        out_type=jax.ShapeDtypeStruct((NC, npad, DEGW), F32),
        mesh=_mesh(),
        compiler_params=pltpu.CompilerParams(use_tc_tiling_on_sc=False),
        scratch_types=[
            pltpu.VMEM((n_chunks, CHUNK), jnp.int32),
            pltpu.VMEM((CHUNK, DEGW), F32),
            pltpu.VMEM((rows_per_tile, DEGW), F32),
            pltpu.VMEM_SHARED((npad, DEGW), F32),
        ],
    )
    def deg_kernel(edges_hbm, deg_out, idx_v, ones_v, stage_v, acc_sh):
        c = lax.axis_index("c")
        s = lax.axis_index("s")

        def zrow(i, _):
            stage_v[i, :] = jnp.zeros((DEGW,), F32)
            return 0
        lax.fori_loop(0, rows_per_tile, zrow, 0)
        pltpu.sync_copy(stage_v, acc_sh.at[pl.ds(s * rows_per_tile, rows_per_tile)])

        def orow(i, _):
            ones_v[i, :] = jnp.full((DEGW,), 1.0, F32)
            return 0
        lax.fori_loop(0, CHUNK, orow, 0)
        pltpu.sync_copy(edges_hbm.at[c, s], idx_v)
        plsc.subcore_barrier()

        def step(j, _):
            pltpu.sync_copy(ones_v, acc_sh.at[idx_v.at[j]], add=True)
            return 0
        lax.fori_loop(0, n_chunks, step, 0)
        plsc.subcore_barrier()
        pltpu.sync_copy(acc_sh.at[pl.ds(s * rows_per_tile, rows_per_tile)], stage_v)
        pltpu.sync_copy(stage_v, deg_out.at[c, pl.ds(s * rows_per_tile, rows_per_tile)])

    return deg_kernel


def _make_agg_kernel(npad, n_chunks, dw):
    nspan = npad // NR
    rows_per_tile = nspan // NS
    wb = 64
    acc_rows = nspan + 8
    trash = nspan

    @functools.partial(
        pl.kernel,
        out_type=jax.ShapeDtypeStruct((NC * npad, dw), F32),
        mesh=_mesh(),
        compiler_params=pltpu.CompilerParams(use_tc_tiling_on_sc=False),
        scratch_types=[
            pltpu.VMEM((n_chunks, CHUNK), jnp.int32),
            pltpu.VMEM((n_chunks, CHUNK), jnp.int32),
            pltpu.VMEM((n_chunks, CHUNK), jnp.int32),
            pltpu.VMEM((CHUNK, dw), F32),
            pltpu.VMEM((CHUNK, dw), F32),
            pltpu.VMEM_SHARED((acc_rows, dw), F32),
            pltpu.SemaphoreType.DMA,
            pltpu.SemaphoreType.DMA,
        ],
    )
    def agg_kernel(xs_hbm, send_hbm, recv_hbm, agg_out,
                   sidx_v, ridx_v, rloc_v, rows_a, rows_b, acc_sh,
                   gsem_a, gsem_b):
        c = lax.axis_index("c")
        s = lax.axis_index("s")

        pltpu.sync_copy(send_hbm.at[s], sidx_v)
        pltpu.sync_copy(recv_hbm.at[s], ridx_v)
        off = jnp.broadcast_to(c * npad, (16,)).astype(jnp.int32)

        def adj(j, _):
            for k in range(CHUNK // 16):
                sl = pl.ds(k * 16, 16)
                sidx_v[j, sl] = sidx_v[j, sl] + off
            return 0
        lax.fori_loop(0, n_chunks, adj, 0)

        def zrow(i, _):
            for k in range(dw // 16):
                rows_a[i, pl.ds(k * 16, 16)] = jnp.zeros((16,), F32)
            if dw % 16:
                rows_a[i, pl.ds(16 * (dw // 16), dw % 16)] = jnp.zeros((dw % 16,), F32)
            return 0
        lax.fori_loop(0, CHUNK, zrow, 0)

        for r in range(NR):
            base = r * nspan
            basev = jnp.broadcast_to(base, (16,)).astype(jnp.int32)
            spanv = jnp.broadcast_to(nspan, (16,)).astype(jnp.int32)
            trashv = jnp.broadcast_to(trash, (16,)).astype(jnp.int32)

            for k in range(rows_per_tile // wb):
                pltpu.sync_copy(
                    rows_a.at[pl.ds(0, wb)],
                    acc_sh.at[pl.ds(s * rows_per_tile + k * wb, wb)])

            @pl.when(s == 0)
            def _():
                pltpu.sync_copy(rows_a.at[pl.ds(0, 8)],
                                acc_sh.at[pl.ds(trash, 8)])

            def loc(j, _):
                for k in range(CHUNK // 16):
                    sl = pl.ds(k * 16, 16)
                    v = ridx_v[j, sl] - basev
                    ok = (v >= 0) & (v < spanv)
                    rloc_v[j, sl] = jnp.where(ok, v, trashv)
                return 0
            lax.fori_loop(0, n_chunks, loc, 0)
            plsc.subcore_barrier()

            pltpu.async_copy(xs_hbm.at[sidx_v.at[0]], rows_a, gsem_a)

            def step(p, _):
                j = 2 * p
                pltpu.make_async_copy(
                    xs_hbm.at[sidx_v.at[j]], rows_a, gsem_a).wait()
                pltpu.async_copy(xs_hbm.at[sidx_v.at[j + 1]], rows_b, gsem_b)
                pltpu.sync_copy(rows_a, acc_sh.at[rloc_v.at[j]], add=True)
                pltpu.make_async_copy(
                    xs_hbm.at[sidx_v.at[j + 1]], rows_b, gsem_b).wait()

                @pl.when(j + 2 < n_chunks)
                def _():
                    pltpu.async_copy(xs_hbm.at[sidx_v.at[j + 2]], rows_a, gsem_a)
                pltpu.sync_copy(rows_b, acc_sh.at[rloc_v.at[j + 1]], add=True)
                return 0
            lax.fori_loop(0, n_chunks // 2, step, 0)
            plsc.subcore_barrier()

            for k in range(rows_per_tile // wb):
                pltpu.sync_copy(
                    acc_sh.at[pl.ds(s * rows_per_tile + k * wb, wb)],
                    rows_a.at[pl.ds(0, wb)])
                pltpu.sync_copy(
                    rows_a.at[pl.ds(0, wb)],
                    agg_out.at[pl.ds(
                        c * npad + base + s * rows_per_tile + k * wb, wb)])

            if r + 1 < NR:
                def zrow2(i, _):
                    for k in range(dw // 16):
                        rows_a[i, pl.ds(k * 16, 16)] = jnp.zeros((16,), F32)
                    if dw % 16:
                        rows_a[i, pl.ds(16 * (dw // 16), dw % 16)] = jnp.zeros((dw % 16,), F32)
                    return 0
                lax.fori_loop(0, wb, zrow2, 0)

    return agg_kernel


def _scale_rows(x2, sdeg, npad, d, blk):
    nblk = npad // blk
    grid = (2 * nblk,)

    def body(x_ref, deg_ref, xs_ref):
        dsv = lax.rsqrt(jnp.maximum(deg_ref[...], 1.0))
        xs_ref[:, :d] = x_ref[...] * dsv
        xs_ref[:, d:] = jnp.broadcast_to(dsv, (blk, SW))

    return pl.pallas_call(
        body,
        grid=grid,
        in_specs=[
            pl.BlockSpec((blk, d), lambda g: (g, 0)),
            pl.BlockSpec((blk, 1), lambda g: (lax.rem(g, nblk), 0)),
        ],
        out_specs=pl.BlockSpec((blk, d + SW), lambda g: (g, 0)),
        out_shape=jax.ShapeDtypeStruct((2 * npad, d + SW), F32),
    )(x2, sdeg)


def _dense_selu(agg2, rdeg, W, b2, npad, d, h, n_true, blk):
    nblk = npad // blk
    grid = (2 * nblk,)

    def body(agg_ref, deg_ref, w_ref, b_ref, nodes_ref, colsum_ref):
        g = pl.program_id(0)
        aggm = agg_ref[...]
        u = jnp.dot(aggm[:, :d], w_ref[...], preferred_element_type=F32)
        dr = lax.rsqrt(jnp.maximum(deg_ref[...], 1.0))
        t = aggm[:, d:d + 1]
        hh = u * dr + (dr * t) * b_ref[...]
        nodes = _SELU_SCALE * jnp.where(hh > 0, hh, _SELU_ALPHA * (lax.exp(hh) - 1.0))
        nodes_ref[...] = nodes

        @pl.when(g == 0)
        def _():
            colsum_ref[...] = jnp.zeros((1, h), F32)

        @pl.when(g < nblk)
        def _():
            row = lax.broadcasted_iota(jnp.int32, (blk, 1), 0) + g * blk
            masked = jnp.where(row < n_true, nodes, jnp.zeros((blk, h), F32))
            colsum_ref[...] += jnp.sum(masked, axis=0, keepdims=True)

    return pl.pallas_call(
        body,
        grid=grid,
        in_specs=[
            pl.BlockSpec((blk, d + SW), lambda g: (g, 0)),
            pl.BlockSpec((blk, 1), lambda g: (lax.rem(g, nblk), 0)),
            pl.BlockSpec((d, h), lambda g: (0, 0)),
            pl.BlockSpec((1, h), lambda g: (0, 0)),
        ],
        out_specs=[
            pl.BlockSpec((blk, h), lambda g: (g, 0)),
            pl.BlockSpec((1, h), lambda g: (0, 0)),
        ],
        out_shape=[
            jax.ShapeDtypeStruct((2 * npad, h), F32),
            jax.ShapeDtypeStruct((1, h), F32),
        ],
    )(agg2, rdeg, W, b2)


def _readout(nodes2, colsum, Wb, npad, h, n_true, blk):
    nblk = npad // blk
    grid = (2 * nblk,)

    def body(nodes_ref, colsum_ref, wb_ref, logits_ref, summary_ref, v_s):
        g = pl.program_id(0)

        @pl.when(g == 0)
        def _():
            sig = jax.nn.sigmoid(colsum_ref[...] * (1.0 / n_true))
            summary_ref[...] = sig
            v_s[...] = jnp.dot(wb_ref[...], jnp.reshape(sig, (h, 1)),
                               preferred_element_type=F32)

        logits_ref[...] = jnp.dot(nodes_ref[...], v_s[...],
                                  preferred_element_type=F32)

    return pl.pallas_call(
        body,
        grid=grid,
        in_specs=[
            pl.BlockSpec((blk, h), lambda g: (g, 0)),
            pl.BlockSpec((1, h), lambda g: (0, 0)),
            pl.BlockSpec((h, h), lambda g: (0, 0)),
        ],
        out_specs=[
            pl.BlockSpec((blk, 1), lambda g: (g, 0)),
            pl.BlockSpec((1, h), lambda g: (0, 0)),
        ],
        out_shape=[
            jax.ShapeDtypeStruct((2 * npad, 1), F32),
            jax.ShapeDtypeStruct((1, h), F32),
        ],
        scratch_shapes=[pltpu.VMEM((h, 1), F32)],
    )(nodes2, colsum, Wb)


def kernel(x, c_x, edge_index, W, b, Wb):
    n, d = x.shape
    h = W.shape[1]
    e = edge_index.shape[1]
    blk = 256
    npad = ((n + 1 + 2 * blk - 1) // (2 * blk)) * (2 * blk)
    echunk = NS * CHUNK * 2
    epad = ((e + echunk - 1) // echunk) * echunk
    n_chunks = epad // (NS * CHUNK)

    senders = edge_index[0].astype(jnp.int32)
    receivers = edge_index[1].astype(jnp.int32)
    pad_e = jnp.full((epad - e,), n, jnp.int32)
    sr = jnp.concatenate([senders, pad_e])
    rr = jnp.concatenate([receivers, pad_e])
    edges3 = jnp.stack([sr, rr]).reshape(2, NS, n_chunks, CHUNK)
    send_g = sr.reshape(NS, n_chunks, CHUNK)
    recv3 = rr.reshape(NS, n_chunks, CHUNK)

    deg2 = _make_degree_kernel(npad, n_chunks)(edges3)
    sdeg = deg2[0, :, 0:1]
    rdeg = deg2[1, :, 0:1]

    x2 = jnp.concatenate([
        jnp.pad(x.astype(F32), ((0, npad - n), (0, 0))),
        jnp.pad(c_x.astype(F32), ((0, npad - n), (0, 0))),
    ])
    xs2 = _scale_rows(x2, sdeg, npad, d, blk)

    agg2 = _make_agg_kernel(npad, n_chunks, d + SW)(xs2, send_g, recv3)

    b2 = jnp.reshape(b.astype(F32), (1, h))
    nodes2, colsum = _dense_selu(agg2, rdeg, W.astype(F32), b2, npad, d, h, n, blk)

    logits2, summary = _readout(nodes2, colsum, Wb.astype(F32), npad, h, n, blk)

    nodes1_o = nodes2[:n]
    nodes2_o = nodes2[npad:npad + n]
    logits = jnp.concatenate([logits2[:n, 0], logits2[npad:npad + n, 0]])
    return (nodes1_o, nodes2_o, summary[0]), logits


# speedup vs baseline: 1.4539x; 1.0671x over previous
import functools

import jax
import jax.numpy as jnp
from jax import lax
from jax.experimental import pallas as pl
from jax.experimental.pallas import tpu as pltpu
from jax.experimental.pallas import tpu_sc as plsc

NC = 2
NS = 16
CHUNK = 80
SW = 16
DEGW = 8
NR = 2
F32 = jnp.float32

_SELU_ALPHA = 1.6732632423543772848170429916717
_SELU_SCALE = 1.0507009873554804934193349852946


def _mesh():
    return plsc.VectorSubcoreMesh(
        core_axis_name="c", subcore_axis_name="s", num_cores=NC, num_subcores=NS)


def _make_degree_kernel(npad, n_chunks):
    rows_per_tile = npad // NS

    @functools.partial(
        pl.kernel,
        out_type=jax.ShapeDtypeStruct((NC, npad, DEGW), F32),
        mesh=_mesh(),
        compiler_params=pltpu.CompilerParams(use_tc_tiling_on_sc=False),
        scratch_types=[
            pltpu.VMEM((n_chunks, CHUNK), jnp.int32),
            pltpu.VMEM((CHUNK, DEGW), F32),
            pltpu.VMEM((rows_per_tile, DEGW), F32),
            pltpu.VMEM_SHARED((npad, DEGW), F32),
        ],
    )
    def deg_kernel(edges_hbm, deg_out, idx_v, ones_v, stage_v, acc_sh):
        c = lax.axis_index("c")
        s = lax.axis_index("s")

        def zrow(i, _):
            stage_v[i, :] = jnp.zeros((DEGW,), F32)
            return 0
        lax.fori_loop(0, rows_per_tile, zrow, 0)
        pltpu.sync_copy(stage_v, acc_sh.at[pl.ds(s * rows_per_tile, rows_per_tile)])

        def orow(i, _):
            ones_v[i, :] = jnp.full((DEGW,), 1.0, F32)
            return 0
        lax.fori_loop(0, CHUNK, orow, 0)
        pltpu.sync_copy(edges_hbm.at[c, s], idx_v)
        plsc.subcore_barrier()

        def step(j, _):
            pltpu.sync_copy(ones_v, acc_sh.at[idx_v.at[j]], add=True)
            return 0
        lax.fori_loop(0, n_chunks, step, 0)
        plsc.subcore_barrier()
        pltpu.sync_copy(acc_sh.at[pl.ds(s * rows_per_tile, rows_per_tile)], stage_v)
        pltpu.sync_copy(stage_v, deg_out.at[c, pl.ds(s * rows_per_tile, rows_per_tile)])

    return deg_kernel


def _make_agg_kernel(npad, n_chunks, dw):
    nspan = npad // NR
    rows_per_tile = nspan // NS
    wb = 64
    acc_rows = nspan + 8
    trash = nspan

    @functools.partial(
        pl.kernel,
        out_type=jax.ShapeDtypeStruct((NC * npad, dw), F32),
        mesh=_mesh(),
        compiler_params=pltpu.CompilerParams(use_tc_tiling_on_sc=False),
        scratch_types=[
            pltpu.VMEM((n_chunks, CHUNK), jnp.int32),
            pltpu.VMEM((n_chunks, CHUNK), jnp.int32),
            pltpu.VMEM((n_chunks, CHUNK), jnp.int32),
            pltpu.VMEM((CHUNK, dw), F32),
            pltpu.VMEM((CHUNK, dw), F32),
            pltpu.VMEM_SHARED((acc_rows, dw), F32),
            pltpu.SemaphoreType.DMA,
            pltpu.SemaphoreType.DMA,
        ],
    )
    def agg_kernel(xs_hbm, send_hbm, recv_hbm, agg_out,
                   sidx_v, ridx_v, rloc_v, rows_a, rows_b, acc_sh,
                   gsem_a, gsem_b):
        c = lax.axis_index("c")
        s = lax.axis_index("s")

        pltpu.sync_copy(send_hbm.at[s], sidx_v)
        pltpu.sync_copy(recv_hbm.at[s], ridx_v)
        off = jnp.broadcast_to(c * npad, (16,)).astype(jnp.int32)

        def adj(j, _):
            for k in range(CHUNK // 16):
                sl = pl.ds(k * 16, 16)
                sidx_v[j, sl] = sidx_v[j, sl] + off
            return 0
        lax.fori_loop(0, n_chunks, adj, 0)

        def zrow(i, _):
            for k in range(dw // 16):
                rows_a[i, pl.ds(k * 16, 16)] = jnp.zeros((16,), F32)
            if dw % 16:
                rows_a[i, pl.ds(16 * (dw // 16), dw % 16)] = jnp.zeros((dw % 16,), F32)
            return 0
        lax.fori_loop(0, CHUNK, zrow, 0)

        for r in range(NR):
            base = r * nspan
            basev = jnp.broadcast_to(base, (16,)).astype(jnp.int32)
            spanv = jnp.broadcast_to(nspan, (16,)).astype(jnp.int32)
            trashv = jnp.broadcast_to(trash, (16,)).astype(jnp.int32)

            for k in range(rows_per_tile // wb):
                pltpu.sync_copy(
                    rows_a.at[pl.ds(0, wb)],
                    acc_sh.at[pl.ds(s * rows_per_tile + k * wb, wb)])

            @pl.when(s == 0)
            def _():
                pltpu.sync_copy(rows_a.at[pl.ds(0, 8)],
                                acc_sh.at[pl.ds(trash, 8)])

            def loc(j, _):
                for k in range(CHUNK // 16):
                    sl = pl.ds(k * 16, 16)
                    v = ridx_v[j, sl] - basev
                    ok = (v >= 0) & (v < spanv)
                    rloc_v[j, sl] = jnp.where(ok, v, trashv)
                return 0
            lax.fori_loop(0, n_chunks, loc, 0)
            plsc.subcore_barrier()

            pltpu.async_copy(xs_hbm.at[sidx_v.at[0]], rows_a, gsem_a)

            def step(p, _):
                j = 2 * p
                pltpu.make_async_copy(
                    xs_hbm.at[sidx_v.at[j]], rows_a, gsem_a).wait()
                pltpu.async_copy(xs_hbm.at[sidx_v.at[j + 1]], rows_b, gsem_b)
                pltpu.sync_copy(rows_a, acc_sh.at[rloc_v.at[j]], add=True)
                pltpu.make_async_copy(
                    xs_hbm.at[sidx_v.at[j + 1]], rows_b, gsem_b).wait()

                @pl.when(j + 2 < n_chunks)
                def _():
                    pltpu.async_copy(xs_hbm.at[sidx_v.at[j + 2]], rows_a, gsem_a)
                pltpu.sync_copy(rows_b, acc_sh.at[rloc_v.at[j + 1]], add=True)
                return 0
            lax.fori_loop(0, n_chunks // 2, step, 0)
            plsc.subcore_barrier()

            for k in range(rows_per_tile // wb):
                pltpu.sync_copy(
                    acc_sh.at[pl.ds(s * rows_per_tile + k * wb, wb)],
                    rows_a.at[pl.ds(0, wb)])
                pltpu.sync_copy(
                    rows_a.at[pl.ds(0, wb)],
                    agg_out.at[pl.ds(
                        c * npad + base + s * rows_per_tile + k * wb, wb)])

            if r + 1 < NR:
                def zrow2(i, _):
                    for k in range(dw // 16):
                        rows_a[i, pl.ds(k * 16, 16)] = jnp.zeros((16,), F32)
                    if dw % 16:
                        rows_a[i, pl.ds(16 * (dw // 16), dw % 16)] = jnp.zeros((dw % 16,), F32)
                    return 0
                lax.fori_loop(0, wb, zrow2, 0)

    return agg_kernel


def _scale_rows(x, c_x, sdeg, npad, d, blk):
    nblk = npad // blk
    grid = (2 * nblk,)

    def body(x_ref, cx_ref, deg_ref, xs_ref):
        g = pl.program_id(0)
        dsv = lax.rsqrt(jnp.maximum(deg_ref[...], 1.0))
        xv = jnp.where(g < nblk, x_ref[...], cx_ref[...])
        xs_ref[:, :d] = xv * dsv
        xs_ref[:, d:] = jnp.broadcast_to(dsv, (blk, SW))

    return pl.pallas_call(
        body,
        grid=grid,
        in_specs=[
            pl.BlockSpec((blk, d), lambda g: (lax.rem(g, nblk), 0)),
            pl.BlockSpec((blk, d), lambda g: (lax.rem(g, nblk), 0)),
            pl.BlockSpec((blk, 1), lambda g: (lax.rem(g, nblk), 0)),
        ],
        out_specs=pl.BlockSpec((blk, d + SW), lambda g: (g, 0)),
        out_shape=jax.ShapeDtypeStruct((2 * npad, d + SW), F32),
    )(x, c_x, sdeg)


def _dense_selu(agg2, rdeg, W, b2, npad, d, h, n_true, blk):
    nblk = npad // blk
    grid = (2 * nblk,)

    def body(agg_ref, deg_ref, w_ref, b_ref, nodes_ref, colsum_ref):
        g = pl.program_id(0)
        aggm = agg_ref[...]
        u = jnp.dot(aggm[:, :d], w_ref[...], preferred_element_type=F32)
        dr = lax.rsqrt(jnp.maximum(deg_ref[...], 1.0))
        t = aggm[:, d:d + 1]
        hh = u * dr + (dr * t) * b_ref[...]
        nodes = _SELU_SCALE * jnp.where(hh > 0, hh, _SELU_ALPHA * (lax.exp(hh) - 1.0))
        nodes_ref[...] = nodes

        @pl.when(g == 0)
        def _():
            colsum_ref[...] = jnp.zeros((1, h), F32)

        @pl.when(g < nblk)
        def _():
            row = lax.broadcasted_iota(jnp.int32, (blk, 1), 0) + g * blk
            masked = jnp.where(row < n_true, nodes, jnp.zeros((blk, h), F32))
            colsum_ref[...] += jnp.sum(masked, axis=0, keepdims=True)

    return pl.pallas_call(
        body,
        grid=grid,
        in_specs=[
            pl.BlockSpec((blk, d + SW), lambda g: (g, 0)),
            pl.BlockSpec((blk, 1), lambda g: (lax.rem(g, nblk), 0)),
            pl.BlockSpec((d, h), lambda g: (0, 0)),
            pl.BlockSpec((1, h), lambda g: (0, 0)),
        ],
        out_specs=[
            pl.BlockSpec((blk, h), lambda g: (g, 0)),
            pl.BlockSpec((1, h), lambda g: (0, 0)),
        ],
        out_shape=[
            jax.ShapeDtypeStruct((2 * npad, h), F32),
            jax.ShapeDtypeStruct((1, h), F32),
        ],
    )(agg2, rdeg, W, b2)


def _readout(nodes2, colsum, Wb, npad, h, n_true, blk):
    nblk = npad // blk
    grid = (2 * nblk,)

    def body(nodes_ref, colsum_ref, wb_ref, logits_ref, summary_ref, v_s):
        g = pl.program_id(0)

        @pl.when(g == 0)
        def _():
            sig = jax.nn.sigmoid(colsum_ref[...] * (1.0 / n_true))
            summary_ref[...] = sig
            v_s[...] = jnp.dot(wb_ref[...], jnp.reshape(sig, (h, 1)),
                               preferred_element_type=F32)

        logits_ref[...] = jnp.dot(nodes_ref[...], v_s[...],
                                  preferred_element_type=F32)

    return pl.pallas_call(
        body,
        grid=grid,
        in_specs=[
            pl.BlockSpec((blk, h), lambda g: (g, 0)),
            pl.BlockSpec((1, h), lambda g: (0, 0)),
            pl.BlockSpec((h, h), lambda g: (0, 0)),
        ],
        out_specs=[
            pl.BlockSpec((blk, 1), lambda g: (g, 0)),
            pl.BlockSpec((1, h), lambda g: (0, 0)),
        ],
        out_shape=[
            jax.ShapeDtypeStruct((2 * npad, 1), F32),
            jax.ShapeDtypeStruct((1, h), F32),
        ],
        scratch_shapes=[pltpu.VMEM((h, 1), F32)],
    )(nodes2, colsum, Wb)


def kernel(x, c_x, edge_index, W, b, Wb):
    n, d = x.shape
    h = W.shape[1]
    e = edge_index.shape[1]
    blk = 512
    npad = ((n + 1 + 2 * blk - 1) // (2 * blk)) * (2 * blk)
    echunk = NS * CHUNK * 2
    epad = ((e + echunk - 1) // echunk) * echunk
    n_chunks = epad // (NS * CHUNK)

    senders = edge_index[0].astype(jnp.int32)
    receivers = edge_index[1].astype(jnp.int32)
    pad_e = jnp.full((epad - e,), n, jnp.int32)
    sr = jnp.concatenate([senders, pad_e])
    rr = jnp.concatenate([receivers, pad_e])
    edges3 = jnp.stack([sr, rr]).reshape(2, NS, n_chunks, CHUNK)
    send_g = sr.reshape(NS, n_chunks, CHUNK)
    recv3 = rr.reshape(NS, n_chunks, CHUNK)

    deg2 = _make_degree_kernel(npad, n_chunks)(edges3)
    sdeg = deg2[0, :, 0:1]
    rdeg = deg2[1, :, 0:1]

    xp = jnp.pad(x.astype(F32), ((0, npad - n), (0, 0)))
    cxp = jnp.pad(c_x.astype(F32), ((0, npad - n), (0, 0)))
    xs2 = _scale_rows(xp, cxp, sdeg, npad, d, blk)

    agg2 = _make_agg_kernel(npad, n_chunks, d + SW)(xs2, send_g, recv3)

    b2 = jnp.reshape(b.astype(F32), (1, h))
    nodes2, colsum = _dense_selu(agg2, rdeg, W.astype(F32), b2, npad, d, h, n, blk)

    logits2, summary = _readout(nodes2, colsum, Wb.astype(F32), npad, h, n, blk)

    nodes1_o = nodes2[:n]
    nodes2_o = nodes2[npad:npad + n]
    logits = jnp.concatenate([logits2[:n, 0], logits2[npad:npad + n, 0]])
    return (nodes1_o, nodes2_o, summary[0]), logits


# speedup vs baseline: 1.5377x; 1.0577x over previous
import functools

import jax
import jax.numpy as jnp
from jax import lax
from jax.experimental import pallas as pl
from jax.experimental.pallas import tpu as pltpu
from jax.experimental.pallas import tpu_sc as plsc

NC = 2
NS = 16
CHUNK = 80
SW = 16
DEGW = 8
NR = 2
F32 = jnp.float32

_SELU_ALPHA = 1.6732632423543772848170429916717
_SELU_SCALE = 1.0507009873554804934193349852946


def _mesh():
    return plsc.VectorSubcoreMesh(
        core_axis_name="c", subcore_axis_name="s", num_cores=NC, num_subcores=NS)


def _make_degree_kernel(npad, n_chunks):
    rows_per_tile = npad // NS

    @functools.partial(
        pl.kernel,
        out_type=jax.ShapeDtypeStruct((NC, npad, DEGW), F32),
        mesh=_mesh(),
        compiler_params=pltpu.CompilerParams(use_tc_tiling_on_sc=False),
        scratch_types=[
            pltpu.VMEM((n_chunks, CHUNK), jnp.int32),
            pltpu.VMEM((CHUNK, DEGW), F32),
            pltpu.VMEM((rows_per_tile, DEGW), F32),
            pltpu.VMEM_SHARED((npad, DEGW), F32),
        ],
    )
    def deg_kernel(edges_hbm, deg_out, idx_v, ones_v, stage_v, acc_sh):
        c = lax.axis_index("c")
        s = lax.axis_index("s")

        def zrow(i, _):
            stage_v[i, :] = jnp.zeros((DEGW,), F32)
            return 0
        lax.fori_loop(0, rows_per_tile, zrow, 0)
        pltpu.sync_copy(stage_v, acc_sh.at[pl.ds(s * rows_per_tile, rows_per_tile)])

        def orow(i, _):
            ones_v[i, :] = jnp.full((DEGW,), 1.0, F32)
            return 0
        lax.fori_loop(0, CHUNK, orow, 0)
        pltpu.sync_copy(edges_hbm.at[c, s], idx_v)
        plsc.subcore_barrier()

        def step(j, _):
            pltpu.sync_copy(ones_v, acc_sh.at[idx_v.at[j]], add=True)
            return 0
        lax.fori_loop(0, n_chunks, step, 0)
        plsc.subcore_barrier()
        pltpu.sync_copy(acc_sh.at[pl.ds(s * rows_per_tile, rows_per_tile)], stage_v)
        pltpu.sync_copy(stage_v, deg_out.at[c, pl.ds(s * rows_per_tile, rows_per_tile)])

    return deg_kernel


def _make_agg_kernel(npad, n_chunks, dw):
    nspan = npad // NR
    rows_per_tile = nspan // NS
    wb = 64
    acc_rows = nspan + 8
    trash = nspan

    @functools.partial(
        pl.kernel,
        out_type=jax.ShapeDtypeStruct((NC * npad, dw), F32),
        mesh=_mesh(),
        compiler_params=pltpu.CompilerParams(use_tc_tiling_on_sc=False),
        scratch_types=[
            pltpu.VMEM((n_chunks, CHUNK), jnp.int32),
            pltpu.VMEM((n_chunks, CHUNK), jnp.int32),
            pltpu.VMEM((n_chunks, CHUNK), jnp.int32),
            pltpu.VMEM((CHUNK, dw), F32),
            pltpu.VMEM((CHUNK, dw), F32),
            pltpu.VMEM_SHARED((acc_rows, dw), F32),
            pltpu.SemaphoreType.DMA,
            pltpu.SemaphoreType.DMA,
        ],
    )
    def agg_kernel(xs_hbm, send_hbm, recv_hbm, agg_out,
                   sidx_v, ridx_v, rloc_v, rows_a, rows_b, acc_sh,
                   gsem_a, gsem_b):
        c = lax.axis_index("c")
        s = lax.axis_index("s")

        pltpu.sync_copy(send_hbm.at[s], sidx_v)
        pltpu.sync_copy(recv_hbm.at[s], ridx_v)
        off = jnp.broadcast_to(c * npad, (16,)).astype(jnp.int32)

        def adj(j, _):
            for k in range(CHUNK // 16):
                sl = pl.ds(k * 16, 16)
                sidx_v[j, sl] = sidx_v[j, sl] + off
            return 0
        lax.fori_loop(0, n_chunks, adj, 0)

        def zrow(i, _):
            for k in range(dw // 16):
                rows_a[i, pl.ds(k * 16, 16)] = jnp.zeros((16,), F32)
            if dw % 16:
                rows_a[i, pl.ds(16 * (dw // 16), dw % 16)] = jnp.zeros((dw % 16,), F32)
            return 0
        lax.fori_loop(0, CHUNK, zrow, 0)

        for r in range(NR):
            base = r * nspan
            basev = jnp.broadcast_to(base, (16,)).astype(jnp.int32)
            spanv = jnp.broadcast_to(nspan, (16,)).astype(jnp.int32)
            trashv = jnp.broadcast_to(trash, (16,)).astype(jnp.int32)

            for k in range(rows_per_tile // wb):
                pltpu.sync_copy(
                    rows_a.at[pl.ds(0, wb)],
                    acc_sh.at[pl.ds(s * rows_per_tile + k * wb, wb)])

            @pl.when(s == 0)
            def _():
                pltpu.sync_copy(rows_a.at[pl.ds(0, 8)],
                                acc_sh.at[pl.ds(trash, 8)])

            def loc(j, _):
                for k in range(CHUNK // 16):
                    sl = pl.ds(k * 16, 16)
                    v = ridx_v[j, sl] - basev
                    ok = (v >= 0) & (v < spanv)
                    rloc_v[j, sl] = jnp.where(ok, v, trashv)
                return 0
            lax.fori_loop(0, n_chunks, loc, 0)
            plsc.subcore_barrier()

            pltpu.async_copy(xs_hbm.at[sidx_v.at[0]], rows_a, gsem_a)

            def step(p, _):
                j = 2 * p
                pltpu.make_async_copy(
                    xs_hbm.at[sidx_v.at[j]], rows_a, gsem_a).wait()
                pltpu.async_copy(xs_hbm.at[sidx_v.at[j + 1]], rows_b, gsem_b)
                pltpu.sync_copy(rows_a, acc_sh.at[rloc_v.at[j]], add=True)
                pltpu.make_async_copy(
                    xs_hbm.at[sidx_v.at[j + 1]], rows_b, gsem_b).wait()

                @pl.when(j + 2 < n_chunks)
                def _():
                    pltpu.async_copy(xs_hbm.at[sidx_v.at[j + 2]], rows_a, gsem_a)
                pltpu.sync_copy(rows_b, acc_sh.at[rloc_v.at[j + 1]], add=True)
                return 0
            lax.fori_loop(0, n_chunks // 2, step, 0)
            plsc.subcore_barrier()

            for k in range(rows_per_tile // wb):
                pltpu.sync_copy(
                    acc_sh.at[pl.ds(s * rows_per_tile + k * wb, wb)],
                    rows_a.at[pl.ds(0, wb)])
                pltpu.sync_copy(
                    rows_a.at[pl.ds(0, wb)],
                    agg_out.at[pl.ds(
                        c * npad + base + s * rows_per_tile + k * wb, wb)])

            if r + 1 < NR:
                def zrow2(i, _):
                    for k in range(dw // 16):
                        rows_a[i, pl.ds(k * 16, 16)] = jnp.zeros((16,), F32)
                    if dw % 16:
                        rows_a[i, pl.ds(16 * (dw // 16), dw % 16)] = jnp.zeros((dw % 16,), F32)
                    return 0
                lax.fori_loop(0, wb, zrow2, 0)

    return agg_kernel


def _scale_rows(x, c_x, sdeg, npad, d, blk):
    nblk = npad // blk
    grid = (2 * nblk,)

    def body(x_ref, cx_ref, deg_ref, xs_ref):
        g = pl.program_id(0)
        dsv = lax.rsqrt(jnp.maximum(deg_ref[...], 1.0))
        xv = jnp.where(g < nblk, x_ref[...], cx_ref[...])
        xs_ref[:, :d] = xv * dsv
        xs_ref[:, d:] = jnp.broadcast_to(dsv, (blk, SW))

    return pl.pallas_call(
        body,
        grid=grid,
        in_specs=[
            pl.BlockSpec((blk, d), lambda g: (lax.rem(g, nblk), 0)),
            pl.BlockSpec((blk, d), lambda g: (lax.rem(g, nblk), 0)),
            pl.BlockSpec((blk, 1), lambda g: (lax.rem(g, nblk), 0)),
        ],
        out_specs=pl.BlockSpec((blk, d + SW), lambda g: (g, 0)),
        out_shape=jax.ShapeDtypeStruct((2 * npad, d + SW), F32),
    )(x, c_x, sdeg)


def _dense_selu(agg2, rdeg, W, b2, npad, d, h, n_true, blk):
    nblk = npad // blk
    grid = (nblk,)
    dw = agg2.shape[1]

    def body(agg_a, agg_b, deg_ref, w_ref, b_ref, n1_ref, n2_ref, colsum_ref):
        g = pl.program_id(0)
        dr = lax.rsqrt(jnp.maximum(deg_ref[...], 1.0))
        wm = w_ref[...]
        bm = b_ref[...]
        row = lax.broadcasted_iota(jnp.int32, (blk, 1), 0) + g * blk
        msk = row < n_true

        def one(aggm):
            u = jnp.dot(aggm[:, :d], wm, preferred_element_type=F32)
            hh = u * dr + (dr * aggm[:, d:d + 1]) * bm
            return _SELU_SCALE * jnp.where(
                hh > 0, hh, _SELU_ALPHA * (lax.exp(hh) - 1.0))

        nodes1 = one(agg_a[...])
        n1_ref[...] = nodes1
        n2_ref[...] = one(agg_b[...])

        @pl.when(g == 0)
        def _():
            colsum_ref[...] = jnp.zeros((1, h), F32)
        colsum_ref[...] += jnp.sum(
            jnp.where(msk, nodes1, jnp.zeros((blk, h), F32)),
            axis=0, keepdims=True)

    return pl.pallas_call(
        body,
        grid=grid,
        in_specs=[
            pl.BlockSpec((blk, dw), lambda g: (g, 0)),
            pl.BlockSpec((blk, dw), lambda g: (g + nblk, 0)),
            pl.BlockSpec((blk, 1), lambda g: (g, 0)),
            pl.BlockSpec((d, h), lambda g: (0, 0)),
            pl.BlockSpec((1, h), lambda g: (0, 0)),
        ],
        out_specs=[
            pl.BlockSpec((blk, h), lambda g: (g, 0)),
            pl.BlockSpec((blk, h), lambda g: (g, 0)),
            pl.BlockSpec((1, h), lambda g: (0, 0)),
        ],
        out_shape=[
            jax.ShapeDtypeStruct((n_true, h), F32),
            jax.ShapeDtypeStruct((n_true, h), F32),
            jax.ShapeDtypeStruct((1, h), F32),
        ],
    )(agg2, agg2, rdeg, W, b2)


def _readout(nodes1, nodes2, colsum, Wb, npad, h, n_true, blk):
    nblk = npad // blk
    grid = (nblk,)

    def body(n1_ref, n2_ref, colsum_ref, wb_ref, l1_ref, l2_ref,
             summary_ref, v_s):
        g = pl.program_id(0)

        @pl.when(g == 0)
        def _():
            sig = jax.nn.sigmoid(colsum_ref[...] * (1.0 / n_true))
            summary_ref[...] = sig
            v_s[...] = jnp.dot(wb_ref[...], jnp.reshape(sig, (h, 1)),
                               preferred_element_type=F32)

        l1_ref[...] = jnp.dot(n1_ref[...], v_s[...],
                              preferred_element_type=F32)
        l2_ref[...] = jnp.dot(n2_ref[...], v_s[...],
                              preferred_element_type=F32)

    return pl.pallas_call(
        body,
        grid=grid,
        in_specs=[
            pl.BlockSpec((blk, h), lambda g: (g, 0)),
            pl.BlockSpec((blk, h), lambda g: (g, 0)),
            pl.BlockSpec((1, h), lambda g: (0, 0)),
            pl.BlockSpec((h, h), lambda g: (0, 0)),
        ],
        out_specs=[
            pl.BlockSpec((blk, 1), lambda g: (g, 0)),
            pl.BlockSpec((blk, 1), lambda g: (g, 0)),
            pl.BlockSpec((1, h), lambda g: (0, 0)),
        ],
        out_shape=[
            jax.ShapeDtypeStruct((n_true, 1), F32),
            jax.ShapeDtypeStruct((n_true, 1), F32),
            jax.ShapeDtypeStruct((1, h), F32),
        ],
        scratch_shapes=[pltpu.VMEM((h, 1), F32)],
    )(nodes1, nodes2, colsum, Wb)


def kernel(x, c_x, edge_index, W, b, Wb):
    n, d = x.shape
    h = W.shape[1]
    e = edge_index.shape[1]
    blk = 512
    npad = ((n + 1 + 2 * blk - 1) // (2 * blk)) * (2 * blk)
    echunk = NS * CHUNK * 2
    epad = ((e + echunk - 1) // echunk) * echunk
    n_chunks = epad // (NS * CHUNK)

    senders = edge_index[0].astype(jnp.int32)
    receivers = edge_index[1].astype(jnp.int32)
    pad_e = jnp.full((epad - e,), n, jnp.int32)
    sr = jnp.concatenate([senders, pad_e])
    rr = jnp.concatenate([receivers, pad_e])
    edges3 = jnp.stack([sr, rr]).reshape(2, NS, n_chunks, CHUNK)
    send_g = sr.reshape(NS, n_chunks, CHUNK)
    recv3 = rr.reshape(NS, n_chunks, CHUNK)

    deg2 = _make_degree_kernel(npad, n_chunks)(edges3)
    sdeg = deg2[0, :, 0:1]
    rdeg = deg2[1, :, 0:1]

    xp = jnp.pad(x.astype(F32), ((0, npad - n), (0, 0)))
    cxp = jnp.pad(c_x.astype(F32), ((0, npad - n), (0, 0)))
    xs2 = _scale_rows(xp, cxp, sdeg, npad, d, blk)

    agg2 = _make_agg_kernel(npad, n_chunks, d + SW)(xs2, send_g, recv3)

    b2 = jnp.reshape(b.astype(F32), (1, h))
    nodes1_o, nodes2_o, colsum = _dense_selu(
        agg2, rdeg, W.astype(F32), b2, npad, d, h, n, blk)

    logits1, logits2, summary = _readout(
        nodes1_o, nodes2_o, colsum, Wb.astype(F32), npad, h, n, blk)

    logits = jnp.concatenate([logits1[:, 0], logits2[:, 0]])
    return (nodes1_o, nodes2_o, summary[0]), logits
